# baseline (speedup 1.0000x reference)
import functools

import jax
import jax.numpy as jnp
import numpy as np
from jax import lax
from jax.experimental import pallas as pl
from jax.experimental.pallas import tpu as pltpu
from jax.experimental.pallas import tpu_sc as plsc

_B = 16384
_F = 128
_H = 256
_C = 91
_K = 45
_FILL = 0.05
_BM = 8192

_NC, _NS, _L = 2, 16, 16
_NW = _NC * _NS
_RW = _B // _NW
_NG = _RW // _L


def _batcher_pairs(n):
    pairs = []
    p = 1
    while p < n:
        k = p
        while k >= 1:
            for j in range(k % p, n - k, 2 * k):
                for i in range(0, min(k, n - j - k)):
                    if (i + j) // (2 * p) == (i + j + k) // (2 * p):
                        pairs.append((i + j, i + j + k))
            k //= 2
        p *= 2
    return pairs


def _selection_network():
    lim = [(a, b) for (a, b) in _batcher_pairs(128) if b < 96]
    needed = {_K - 1}
    kept = []
    for (a, b) in reversed(lim):
        if a in needed or b in needed:
            kept.append((a, b))
            needed.add(a)
            needed.add(b)
    kept.reverse()
    return kept

_NET = _selection_network()


def _mlp_kernel(x_ref, w1_ref, b1_ref, w2_ref, b2_ref, w3_ref, b3_ref,
                w4_ref, b4_ref, w4r_ref, b4t_ref, logits_ref, lgt_ref):
    x = x_ref[...]
    h = jnp.maximum(jnp.dot(x, w1_ref[...], preferred_element_type=jnp.float32)
                    + b1_ref[...], 0.0)
    h = jnp.maximum(jnp.dot(h, w2_ref[...], preferred_element_type=jnp.float32)
                    + b2_ref[...], 0.0)
    h = jnp.maximum(jnp.dot(h, w3_ref[...], preferred_element_type=jnp.float32)
                    + b3_ref[...], 0.0)
    logits_ref[...] = jnp.dot(h, w4_ref[...],
                              preferred_element_type=jnp.float32) + b4_ref[...]
    lgt_ref[...] = jax.lax.dot_general(
        w4r_ref[...], h, (((1,), (1,)), ((), ())),
        preferred_element_type=jnp.float32) + b4t_ref[...]


def _sc_topk(lgt_hbm, out_hbm, lg_v, out_v):
    wid = lax.axis_index("s") * _NC + lax.axis_index("c")
    col0 = wid * _RW
    pltpu.sync_copy(lgt_hbm.at[:, pl.ds(col0 * 1, _RW)], lg_v)

    inf_vec = jnp.full((_L,), np.float32(np.inf), dtype=jnp.float32)

    def group(g, carry):
        rb = g * _L
        vs = [lg_v[c, pl.ds(rb, _L)] for c in range(_C)]
        w = vs + [inf_vec] * (96 - _C)
        for (a, b) in _NET:
            wa, wb = w[a], w[b]
            w[a] = jnp.minimum(wa, wb)
            w[b] = jnp.maximum(wa, wb)
        t = w[_K - 1]

        m = jnp.zeros((_L,), jnp.int32)
        for c in range(_C):
            m = m + jnp.where(vs[c] < t, 1, 0)
        need = jnp.int32(_K) - m

        taken = jnp.zeros((_L,), jnp.int32)
        rt0 = jnp.zeros((_L,), jnp.float32)
        rt1 = jnp.zeros((_L,), jnp.float32)
        for c in range(_C):
            u = vs[c]
            take = (u == t) & (taken < need)
            taken = taken + jnp.where(take, 1, 0)
            mv = jnp.where((u < t) | take, jnp.float32(_FILL), u)
            if c % 2 == 0:
                rt0 = rt0 + mv
            else:
                rt1 = rt1 + mv
        out_v[0, pl.ds(rb, _L)] = rt0
        out_v[1, pl.ds(rb, _L)] = rt1
        return carry

    lax.fori_loop(0, _NG, group, 0)
    pltpu.sync_copy(out_v, out_hbm.at[:, pl.ds(col0 * 1, _RW)])


@jax.jit
def _run(x, w1t, b1, w2t, b2, w3t, b3, w4t, b4, w4r, b4t):
    grid = (_B // _BM,)
    logits, lgt = pl.pallas_call(
        _mlp_kernel,
        grid=grid,
        in_specs=[
            pl.BlockSpec((_BM, _F), lambda i: (i, 0)),
            pl.BlockSpec((_F, _H), lambda i: (0, 0)),
            pl.BlockSpec((1, _H), lambda i: (0, 0)),
            pl.BlockSpec((_H, _H), lambda i: (0, 0)),
            pl.BlockSpec((1, _H), lambda i: (0, 0)),
            pl.BlockSpec((_H, _H), lambda i: (0, 0)),
            pl.BlockSpec((1, _H), lambda i: (0, 0)),
            pl.BlockSpec((_H, _C), lambda i: (0, 0)),
            pl.BlockSpec((1, _C), lambda i: (0, 0)),
            pl.BlockSpec((_C, _H), lambda i: (0, 0)),
            pl.BlockSpec((_C, 1), lambda i: (0, 0)),
        ],
        out_specs=[
            pl.BlockSpec((_BM, _C), lambda i: (i, 0)),
            pl.BlockSpec((_C, _BM), lambda i: (0, i)),
        ],
        out_shape=[
            jax.ShapeDtypeStruct((_B, _C), jnp.float32),
            jax.ShapeDtypeStruct((_C, _B), jnp.float32),
        ],
    )(x, w1t, b1, w2t, b2, w3t, b3, w4t, b4, w4r, b4t)

    sc = functools.partial(
        pl.kernel,
        mesh=plsc.VectorSubcoreMesh(core_axis_name="c", subcore_axis_name="s"),
        out_type=jax.ShapeDtypeStruct((2, _B), jnp.float32),
        scratch_types=[
            pltpu.VMEM((_C, _RW), jnp.float32),
            pltpu.VMEM((2, _RW), jnp.float32),
        ],
    )(_sc_topk)
    rtt = sc(lgt)
    return logits, rtt


def kernel(x_vector, W1, b1, W2, b2, W3, b3, W4, b4, bn_gamma, bn_beta,
           bn_mean, bn_var, topic_course):
    eps = 1e-5
    scale = bn_gamma * jax.lax.rsqrt(bn_var + eps)
    shift = bn_beta - bn_mean * scale
    w1t = (W1 * scale[:, None]).T
    b1f = (b1 * scale + shift)[None, :]
    w2t = (W2 * scale[:, None]).T
    b2f = (b2 * scale + shift)[None, :]
    w3t = (W3 * scale[:, None]).T
    b3f = (b3 * scale + shift)[None, :]
    w4t = W4.T
    b4f = b4[None, :]
    b4t = b4[:, None]
    logits, rtt = _run(x_vector, w1t, b1f, w2t, b2f, w3t, b3f, w4t, b4f,
                       W4, b4t)
    return (logits, rtt.T)


# speedup vs baseline: 1.0076x; 1.0076x over previous
import functools

import jax
import jax.numpy as jnp
import numpy as np
from jax import lax
from jax.experimental import pallas as pl
from jax.experimental.pallas import tpu as pltpu
from jax.experimental.pallas import tpu_sc as plsc

_B = 16384
_F = 128
_H = 256
_C = 91
_K = 45
_FILL = 0.05
_BM = 8192

_NC, _NS, _L = 2, 16, 16
_NW = _NC * _NS
_RW = _B // _NW
_NG = _RW // _L


def _batcher_pairs(n):
    pairs = []
    p = 1
    while p < n:
        k = p
        while k >= 1:
            for j in range(k % p, n - k, 2 * k):
                for i in range(0, min(k, n - j - k)):
                    if (i + j) // (2 * p) == (i + j + k) // (2 * p):
                        pairs.append((i + j, i + j + k))
            k //= 2
        p *= 2
    return pairs


def _selection_network():
    lim = [(a, b) for (a, b) in _batcher_pairs(128) if b < 96]
    needed = {_K - 1}
    kept = []
    for (a, b) in reversed(lim):
        if a in needed or b in needed:
            kept.append((a, b))
            needed.add(a)
            needed.add(b)
    kept.reverse()
    return kept

_NET = _selection_network()


def _mlp_kernel(x_ref, w1_ref, b1_ref, w2_ref, b2_ref, w3_ref, b3_ref,
                w4_ref, b4_ref, w4r_ref, b4t_ref, logits_ref, lgt_ref):
    x = x_ref[...]
    h = jnp.maximum(jnp.dot(x, w1_ref[...], preferred_element_type=jnp.float32)
                    + b1_ref[...], 0.0)
    h = jnp.maximum(jnp.dot(h, w2_ref[...], preferred_element_type=jnp.float32)
                    + b2_ref[...], 0.0)
    h = jnp.maximum(jnp.dot(h, w3_ref[...], preferred_element_type=jnp.float32)
                    + b3_ref[...], 0.0)
    logits_ref[...] = jnp.dot(h, w4_ref[...],
                              preferred_element_type=jnp.float32) + b4_ref[...]
    lgt_ref[...] = jax.lax.dot_general(
        w4r_ref[...], h, (((1,), (1,)), ((), ())),
        preferred_element_type=jnp.float32) + b4t_ref[...]


def _sc_topk(lgt_hbm, out_hbm, lg_v, out_v):
    wid = lax.axis_index("s") * _NC + lax.axis_index("c")
    col0 = wid * _RW
    pltpu.sync_copy(lgt_hbm.at[:, pl.ds(col0 * 1, _RW)], lg_v)

    inf_vec = jnp.full((_L,), np.float32(np.inf), dtype=jnp.float32)

    def group(g, carry):
        rb = g * _L
        w = [lg_v[c, pl.ds(rb, _L)] for c in range(_C)]
        w += [inf_vec] * (96 - _C)
        for (a, b) in _NET:
            wa, wb = w[a], w[b]
            w[a] = jnp.minimum(wa, wb)
            w[b] = jnp.maximum(wa, wb)
        t = w[_K - 1]

        m = jnp.zeros((_L,), jnp.int32)
        for c in range(_C):
            m = m + jnp.where(lg_v[c, pl.ds(rb, _L)] < t, 1, 0)
        need = jnp.int32(_K) - m

        taken = jnp.zeros((_L,), jnp.int32)
        rt0 = jnp.zeros((_L,), jnp.float32)
        rt1 = jnp.zeros((_L,), jnp.float32)
        for c in range(_C):
            u = lg_v[c, pl.ds(rb, _L)]
            take = (u == t) & (taken < need)
            taken = taken + jnp.where(take, 1, 0)
            mv = jnp.where((u < t) | take, jnp.float32(_FILL), u)
            if c % 2 == 0:
                rt0 = rt0 + mv
            else:
                rt1 = rt1 + mv
        out_v[0, pl.ds(rb, _L)] = rt0
        out_v[1, pl.ds(rb, _L)] = rt1
        return carry

    lax.fori_loop(0, _NG, group, 0)
    pltpu.sync_copy(out_v, out_hbm.at[:, pl.ds(col0 * 1, _RW)])


@jax.jit
def _run(x, w1t, b1, w2t, b2, w3t, b3, w4t, b4, w4r, b4t):
    grid = (_B // _BM,)
    logits, lgt = pl.pallas_call(
        _mlp_kernel,
        grid=grid,
        in_specs=[
            pl.BlockSpec((_BM, _F), lambda i: (i, 0)),
            pl.BlockSpec((_F, _H), lambda i: (0, 0)),
            pl.BlockSpec((1, _H), lambda i: (0, 0)),
            pl.BlockSpec((_H, _H), lambda i: (0, 0)),
            pl.BlockSpec((1, _H), lambda i: (0, 0)),
            pl.BlockSpec((_H, _H), lambda i: (0, 0)),
            pl.BlockSpec((1, _H), lambda i: (0, 0)),
            pl.BlockSpec((_H, _C), lambda i: (0, 0)),
            pl.BlockSpec((1, _C), lambda i: (0, 0)),
            pl.BlockSpec((_C, _H), lambda i: (0, 0)),
            pl.BlockSpec((_C, 1), lambda i: (0, 0)),
        ],
        out_specs=[
            pl.BlockSpec((_BM, _C), lambda i: (i, 0)),
            pl.BlockSpec((_C, _BM), lambda i: (0, i)),
        ],
        out_shape=[
            jax.ShapeDtypeStruct((_B, _C), jnp.float32),
            jax.ShapeDtypeStruct((_C, _B), jnp.float32),
        ],
    )(x, w1t, b1, w2t, b2, w3t, b3, w4t, b4, w4r, b4t)

    sc = functools.partial(
        pl.kernel,
        mesh=plsc.VectorSubcoreMesh(core_axis_name="c", subcore_axis_name="s"),
        out_type=jax.ShapeDtypeStruct((2, _B), jnp.float32),
        scratch_types=[
            pltpu.VMEM((_C, _RW), jnp.float32),
            pltpu.VMEM((2, _RW), jnp.float32),
        ],
    )(_sc_topk)
    rtt = sc(lgt)
    return logits, rtt


def kernel(x_vector, W1, b1, W2, b2, W3, b3, W4, b4, bn_gamma, bn_beta,
           bn_mean, bn_var, topic_course):
    eps = 1e-5
    scale = bn_gamma * jax.lax.rsqrt(bn_var + eps)
    shift = bn_beta - bn_mean * scale
    w1t = (W1 * scale[:, None]).T
    b1f = (b1 * scale + shift)[None, :]
    w2t = (W2 * scale[:, None]).T
    b2f = (b2 * scale + shift)[None, :]
    w3t = (W3 * scale[:, None]).T
    b3f = (b3 * scale + shift)[None, :]
    w4t = W4.T
    b4f = b4[None, :]
    b4t = b4[:, None]
    logits, rtt = _run(x_vector, w1t, b1f, w2t, b2f, w3t, b3f, w4t, b4f,
                       W4, b4t)
    return (logits, rtt.T)


# speedup vs baseline: 1.0963x; 1.0881x over previous
import jax
import jax.numpy as jnp
import numpy as np
from jax.experimental import pallas as pl

_B = 16384
_F = 128
_H = 256
_C = 91
_K = 45
_FILL = 0.05
_BM = 8192


def _fused_kernel(x_ref, w1_ref, b1_ref, w2_ref, b2_ref, w3_ref, b3_ref,
                  w4_ref, b4_ref, w4r_ref, b4t_ref, ones_ref, slt_ref,
                  tc_ref, logits_ref, rtt_ref):
    x = x_ref[...]
    h = jnp.maximum(jnp.dot(x, w1_ref[...], preferred_element_type=jnp.float32)
                    + b1_ref[...], 0.0)
    h = jnp.maximum(jnp.dot(h, w2_ref[...], preferred_element_type=jnp.float32)
                    + b2_ref[...], 0.0)
    h = jnp.maximum(jnp.dot(h, w3_ref[...], preferred_element_type=jnp.float32)
                    + b3_ref[...], 0.0)
    logits_ref[...] = jnp.dot(h, w4_ref[...],
                              preferred_element_type=jnp.float32) + b4_ref[...]

    lgt = jax.lax.dot_general(w4r_ref[...], h, (((1,), (1,)), ((), ())),
                              preferred_element_type=jnp.float32) + b4t_ref[...]

    ones_row = ones_ref[...]
    ones_bf = ones_row.astype(jnp.bfloat16)
    one_bf = jnp.bfloat16(1.0)
    zero_bf = jnp.bfloat16(0.0)

    def count_lt16(vals16, c16, w):
        cmpb = jnp.where(vals16 < c16, one_bf, zero_bf)
        return jnp.dot(ones_bf, cmpb, preferred_element_type=jnp.float32)

    def descend16(vals16, kvec, w):
        p = jnp.full((1, w), -32768, dtype=jnp.int32)
        for b in range(15, -1, -1):
            c = p + (1 << b)
            cnt = count_lt16(vals16, c.astype(jnp.int16), w)
            p = jnp.where(cnt >= kvec, p, c)
        return p

    _W = 1024
    for s in range(_BM // _W):
        lgs = lgt[:, s * _W:(s + 1) * _W]
        si = jax.lax.bitcast_convert_type(lgs, jnp.int32)
        sm = jnp.where(si < 0, si ^ jnp.int32(0x7FFFFFFF), si)
        hi = (sm >> 16).astype(jnp.int16)
        lo = ((sm & jnp.int32(0xFFFF)) - 32768).astype(jnp.int16)

        k1 = jnp.full((1, _W), float(_K), dtype=jnp.float32)
        p1 = descend16(hi, k1, _W)
        p1_16 = p1.astype(jnp.int16)
        m1 = count_lt16(hi, p1_16, _W)
        eligible = hi == p1_16
        val2 = jnp.where(eligible, lo, jnp.int16(32767))
        p2 = descend16(val2, k1 - m1, _W)
        p = (p1 << 16) + (p2 + 32768)

        lt = sm < p
        ltf = jnp.where(lt, 1.0, 0.0)
        m = jnp.dot(ones_row, ltf, preferred_element_type=jnp.float32)
        eq = sm == p
        eqf = jnp.where(eq, 1.0, 0.0)
        pe = jnp.dot(slt_ref[...], eqf, preferred_element_type=jnp.float32)
        sel = lt | (eq & (pe < (float(_K) - m)))
        maskedt = jnp.where(sel, _FILL, lgs)
        rtt_ref[:, s * _W:(s + 1) * _W] = jnp.dot(
            tc_ref[...], maskedt, preferred_element_type=jnp.float32)


@jax.jit
def _run(x, w1t, b1, w2t, b2, w3t, b3, w4t, b4, w4r, b4t, ones_row, slt, tc):
    grid = (_B // _BM,)
    return pl.pallas_call(
        _fused_kernel,
        grid=grid,
        in_specs=[
            pl.BlockSpec((_BM, _F), lambda i: (i, 0)),
            pl.BlockSpec((_F, _H), lambda i: (0, 0)),
            pl.BlockSpec((1, _H), lambda i: (0, 0)),
            pl.BlockSpec((_H, _H), lambda i: (0, 0)),
            pl.BlockSpec((1, _H), lambda i: (0, 0)),
            pl.BlockSpec((_H, _H), lambda i: (0, 0)),
            pl.BlockSpec((1, _H), lambda i: (0, 0)),
            pl.BlockSpec((_H, _C), lambda i: (0, 0)),
            pl.BlockSpec((1, _C), lambda i: (0, 0)),
            pl.BlockSpec((_C, _H), lambda i: (0, 0)),
            pl.BlockSpec((_C, 1), lambda i: (0, 0)),
            pl.BlockSpec((1, _C), lambda i: (0, 0)),
            pl.BlockSpec((_C, _C), lambda i: (0, 0)),
            pl.BlockSpec((2, _C), lambda i: (0, 0)),
        ],
        out_specs=[
            pl.BlockSpec((_BM, _C), lambda i: (i, 0)),
            pl.BlockSpec((2, _BM), lambda i: (0, i)),
        ],
        out_shape=[
            jax.ShapeDtypeStruct((_B, _C), jnp.float32),
            jax.ShapeDtypeStruct((2, _B), jnp.float32),
        ],
    )(x, w1t, b1, w2t, b2, w3t, b3, w4t, b4, w4r, b4t, ones_row, slt, tc)


def kernel(x_vector, W1, b1, W2, b2, W3, b3, W4, b4, bn_gamma, bn_beta,
           bn_mean, bn_var, topic_course):
    eps = 1e-5
    scale = bn_gamma * jax.lax.rsqrt(bn_var + eps)
    shift = bn_beta - bn_mean * scale
    w1t = (W1 * scale[:, None]).T
    b1f = (b1 * scale + shift)[None, :]
    w2t = (W2 * scale[:, None]).T
    b2f = (b2 * scale + shift)[None, :]
    w3t = (W3 * scale[:, None]).T
    b3f = (b3 * scale + shift)[None, :]
    w4t = W4.T
    b4f = b4[None, :]
    b4t = b4[:, None]
    ones_row = jnp.ones((1, _C), dtype=jnp.float32)
    slt = jnp.asarray(np.tril(np.ones((_C, _C), dtype=np.float32), k=-1))
    logits, rtt = _run(x_vector, w1t, b1f, w2t, b2f, w3t, b3f, w4t, b4f,
                       W4, b4t, ones_row, slt, topic_course)
    return (logits, rtt.T)


# speedup vs baseline: 2.1854x; 1.9934x over previous
import jax
import jax.numpy as jnp
import numpy as np
from jax.experimental import pallas as pl

_B = 16384
_F = 128
_H = 256
_C = 91
_K = 45
_FILL = 0.05
_BM = 8192


def _fused_kernel(x_ref, w1_ref, b1_ref, w2_ref, b2_ref, w3_ref, b3_ref,
                  w4_ref, b4_ref, w4r_ref, b4t_ref, ones_ref, slt_ref,
                  tc_ref, logits_ref, rtt_ref):
    x = x_ref[...]
    h = jnp.maximum(jnp.dot(x, w1_ref[...], preferred_element_type=jnp.float32)
                    + b1_ref[...], 0.0)
    h = jnp.maximum(jnp.dot(h, w2_ref[...], preferred_element_type=jnp.float32)
                    + b2_ref[...], 0.0)
    h = jnp.maximum(jnp.dot(h, w3_ref[...], preferred_element_type=jnp.float32)
                    + b3_ref[...], 0.0)
    logits_ref[...] = jnp.dot(h, w4_ref[...],
                              preferred_element_type=jnp.float32) + b4_ref[...]

    lgt = jax.lax.dot_general(w4r_ref[...], h, (((1,), (1,)), ((), ())),
                              preferred_element_type=jnp.float32) + b4t_ref[...]

    si = jax.lax.bitcast_convert_type(lgt, jnp.int32)
    sm = jnp.where(si < 0, si ^ jnp.int32(0x7FFFFFFF), si)

    ones_row = ones_ref[...]
    p = jnp.full((1, _BM), np.int32(-2**31), dtype=jnp.int32)
    for b in range(31, -1, -1):
        bit = np.int32((1 << b) if b < 31 else -(1 << 31))
        c = p + bit
        cmpf = jnp.where(sm < c, 1.0, 0.0)
        cnt = jnp.dot(ones_row, cmpf, preferred_element_type=jnp.float32)
        p = jnp.where(cnt >= float(_K), p, c)

    lt = sm < p
    ltf = jnp.where(lt, 1.0, 0.0)
    m = jnp.dot(ones_row, ltf, preferred_element_type=jnp.float32)
    eq = sm == p
    eqf = jnp.where(eq, 1.0, 0.0)
    pe = jnp.dot(slt_ref[...], eqf, preferred_element_type=jnp.float32)
    sel = lt | (eq & (pe < (float(_K) - m)))
    maskedt = jnp.where(sel, _FILL, lgt)
    rtt_ref[...] = jnp.dot(tc_ref[...], maskedt,
                           preferred_element_type=jnp.float32)


@jax.jit
def _run(x, w1t, b1, w2t, b2, w3t, b3, w4t, b4, w4r, b4t, ones_row, slt, tc):
    grid = (_B // _BM,)
    return pl.pallas_call(
        _fused_kernel,
        grid=grid,
        in_specs=[
            pl.BlockSpec((_BM, _F), lambda i: (i, 0)),
            pl.BlockSpec((_F, _H), lambda i: (0, 0)),
            pl.BlockSpec((1, _H), lambda i: (0, 0)),
            pl.BlockSpec((_H, _H), lambda i: (0, 0)),
            pl.BlockSpec((1, _H), lambda i: (0, 0)),
            pl.BlockSpec((_H, _H), lambda i: (0, 0)),
            pl.BlockSpec((1, _H), lambda i: (0, 0)),
            pl.BlockSpec((_H, _C), lambda i: (0, 0)),
            pl.BlockSpec((1, _C), lambda i: (0, 0)),
            pl.BlockSpec((_C, _H), lambda i: (0, 0)),
            pl.BlockSpec((_C, 1), lambda i: (0, 0)),
            pl.BlockSpec((1, _C), lambda i: (0, 0)),
            pl.BlockSpec((_C, _C), lambda i: (0, 0)),
            pl.BlockSpec((2, _C), lambda i: (0, 0)),
        ],
        out_specs=[
            pl.BlockSpec((_BM, _C), lambda i: (i, 0)),
            pl.BlockSpec((2, _BM), lambda i: (0, i)),
        ],
        out_shape=[
            jax.ShapeDtypeStruct((_B, _C), jnp.float32),
            jax.ShapeDtypeStruct((2, _B), jnp.float32),
        ],
    )(x, w1t, b1, w2t, b2, w3t, b3, w4t, b4, w4r, b4t, ones_row, slt, tc)


def kernel(x_vector, W1, b1, W2, b2, W3, b3, W4, b4, bn_gamma, bn_beta,
           bn_mean, bn_var, topic_course):
    eps = 1e-5
    scale = bn_gamma * jax.lax.rsqrt(bn_var + eps)
    shift = bn_beta - bn_mean * scale
    w1t = (W1 * scale[:, None]).T
    b1f = (b1 * scale + shift)[None, :]
    w2t = (W2 * scale[:, None]).T
    b2f = (b2 * scale + shift)[None, :]
    w3t = (W3 * scale[:, None]).T
    b3f = (b3 * scale + shift)[None, :]
    w4t = W4.T
    b4f = b4[None, :]
    b4t = b4[:, None]
    ones_row = jnp.ones((1, _C), dtype=jnp.float32)
    slt = jnp.asarray(np.tril(np.ones((_C, _C), dtype=np.float32), k=-1))
    logits, rtt = _run(x_vector, w1t, b1f, w2t, b2f, w3t, b3f, w4t, b4f,
                       W4, b4t, ones_row, slt, topic_course)
    return (logits, rtt.T)


# speedup vs baseline: 2.2309x; 1.0208x over previous
import jax
import jax.numpy as jnp
import numpy as np
from jax.experimental import pallas as pl

_B = 16384
_F = 128
_H = 256
_C = 91
_K = 45
_FILL = 0.05
_BM = 8192


def _fused_kernel(x_ref, w1_ref, b1_ref, w2_ref, b2_ref, w3_ref, b3_ref,
                  w4_ref, b4_ref, w4r_ref, b4t_ref, ones_ref, slt_ref,
                  tc_ref, logits_ref, rtt_ref):
    x = x_ref[...]
    h = jnp.maximum(jnp.dot(x, w1_ref[...], preferred_element_type=jnp.float32)
                    + b1_ref[...], 0.0)
    h = jnp.maximum(jnp.dot(h, w2_ref[...], preferred_element_type=jnp.float32)
                    + b2_ref[...], 0.0)
    h = jnp.maximum(jnp.dot(h, w3_ref[...], preferred_element_type=jnp.float32)
                    + b3_ref[...], 0.0)
    logits_ref[...] = jnp.dot(h, w4_ref[...],
                              preferred_element_type=jnp.float32) + b4_ref[...]

    lgt = jax.lax.dot_general(w4r_ref[...], h, (((1,), (1,)), ((), ())),
                              preferred_element_type=jnp.float32) + b4t_ref[...]

    si = jax.lax.bitcast_convert_type(lgt, jnp.int32)
    sm = jnp.where(si < 0, si ^ jnp.int32(0x7FFFFFFF), si)

    hi = (sm >> 16).astype(jnp.int16)
    lo = ((sm & jnp.int32(0xFFFF)) - 32768).astype(jnp.int16)

    ones_row = ones_ref[...]
    ones_bf = ones_row.astype(jnp.bfloat16)
    one_bf = jnp.bfloat16(1.0)
    zero_bf = jnp.bfloat16(0.0)

    def count_lt16(vals16, c16):
        cmpb = jnp.where(vals16 < c16, one_bf, zero_bf)
        return jnp.dot(ones_bf, cmpb, preferred_element_type=jnp.float32)

    def descend16(vals16, kvec):
        p = jnp.full((1, _BM), -32768, dtype=jnp.int32)
        for b in range(15, -1, -1):
            c = p + (1 << b)
            cnt = count_lt16(vals16, c.astype(jnp.int16))
            p = jnp.where(cnt >= kvec, p, c)
        return p

    k1 = jnp.full((1, _BM), float(_K), dtype=jnp.float32)
    p1 = descend16(hi, k1)
    p1_16 = p1.astype(jnp.int16)
    m1 = count_lt16(hi, p1_16)
    eligible = hi == p1_16
    val2 = jnp.where(eligible, lo, jnp.int16(32767))
    p2 = descend16(val2, k1 - m1)
    p = (p1 << 16) + (p2 + 32768)

    lt = sm < p
    ltf = jnp.where(lt, 1.0, 0.0)
    m = jnp.dot(ones_row, ltf, preferred_element_type=jnp.float32)
    eq = sm == p
    eqf = jnp.where(eq, 1.0, 0.0)
    pe = jnp.dot(slt_ref[...], eqf, preferred_element_type=jnp.float32)
    sel = lt | (eq & (pe < (float(_K) - m)))
    maskedt = jnp.where(sel, _FILL, lgt)
    rtt_ref[...] = jnp.dot(tc_ref[...], maskedt,
                           preferred_element_type=jnp.float32)


@jax.jit
def _run(x, w1t, b1, w2t, b2, w3t, b3, w4t, b4, w4r, b4t, ones_row, slt, tc):
    grid = (_B // _BM,)
    return pl.pallas_call(
        _fused_kernel,
        grid=grid,
        in_specs=[
            pl.BlockSpec((_BM, _F), lambda i: (i, 0)),
            pl.BlockSpec((_F, _H), lambda i: (0, 0)),
            pl.BlockSpec((1, _H), lambda i: (0, 0)),
            pl.BlockSpec((_H, _H), lambda i: (0, 0)),
            pl.BlockSpec((1, _H), lambda i: (0, 0)),
            pl.BlockSpec((_H, _H), lambda i: (0, 0)),
            pl.BlockSpec((1, _H), lambda i: (0, 0)),
            pl.BlockSpec((_H, _C), lambda i: (0, 0)),
            pl.BlockSpec((1, _C), lambda i: (0, 0)),
            pl.BlockSpec((_C, _H), lambda i: (0, 0)),
            pl.BlockSpec((_C, 1), lambda i: (0, 0)),
            pl.BlockSpec((1, _C), lambda i: (0, 0)),
            pl.BlockSpec((_C, _C), lambda i: (0, 0)),
            pl.BlockSpec((2, _C), lambda i: (0, 0)),
        ],
        out_specs=[
            pl.BlockSpec((_BM, _C), lambda i: (i, 0)),
            pl.BlockSpec((2, _BM), lambda i: (0, i)),
        ],
        out_shape=[
            jax.ShapeDtypeStruct((_B, _C), jnp.float32),
            jax.ShapeDtypeStruct((2, _B), jnp.float32),
        ],
    )(x, w1t, b1, w2t, b2, w3t, b3, w4t, b4, w4r, b4t, ones_row, slt, tc)


def kernel(x_vector, W1, b1, W2, b2, W3, b3, W4, b4, bn_gamma, bn_beta,
           bn_mean, bn_var, topic_course):
    eps = 1e-5
    scale = bn_gamma * jax.lax.rsqrt(bn_var + eps)
    shift = bn_beta - bn_mean * scale
    w1t = (W1 * scale[:, None]).T
    b1f = (b1 * scale + shift)[None, :]
    w2t = (W2 * scale[:, None]).T
    b2f = (b2 * scale + shift)[None, :]
    w3t = (W3 * scale[:, None]).T
    b3f = (b3 * scale + shift)[None, :]
    w4t = W4.T
    b4f = b4[None, :]
    b4t = b4[:, None]
    ones_row = jnp.ones((1, _C), dtype=jnp.float32)
    slt = jnp.asarray(np.tril(np.ones((_C, _C), dtype=np.float32), k=-1))
    logits, rtt = _run(x_vector, w1t, b1f, w2t, b2f, w3t, b3f, w4t, b4f,
                       W4, b4t, ones_row, slt, topic_course)
    return (logits, rtt.T)
